# baseline (device time: 261784 ns/iter reference)
import jax
import jax.numpy as jnp
from jax import lax
from jax.experimental import pallas as pl
from jax.experimental.pallas import tpu as pltpu

N_DEV = 4
N_LOCAL_E = 8
N_EXPERTS = N_DEV * N_LOCAL_E
CAP = 128
N_SLOTS = N_DEV * N_LOCAL_E * CAP

_VMEM = 64 * 1024 * 1024


def _prep_body(x_ref, rw_ref, ri_ref, ohT_ref, disp_ref, p_ref, xbf_ref):
    xb = x_ref[...].astype(jnp.bfloat16)
    xbf_ref[...] = xb
    disp_ref[...] = jnp.dot(
        ohT_ref[...], xb, preferred_element_type=jnp.float32
    ).astype(jnp.bfloat16)

    scores = jnp.dot(x_ref[...], rw_ref[...], preferred_element_type=jnp.float32)
    m = jnp.max(scores, axis=-1, keepdims=True)
    ex = jnp.exp(scores - m)
    probs = ex / jnp.sum(ex, axis=-1, keepdims=True)
    eid = lax.broadcasted_iota(jnp.int32, scores.shape, 1)
    sel = (eid == ri_ref[...]).astype(jnp.float32)
    p_ref[...] = jnp.sum(probs * sel, axis=-1, keepdims=True)


def _moe_body(disp_ref, x_ref, sw_ref, w_ref,
              yout_ref, shared_ref,
              recv_ref, send_sem, recv_sem, send_sem2, recv_sem2):
    me = lax.axis_index("i")

    bsem = pltpu.get_barrier_semaphore()
    for j in range(1, N_DEV):
        pl.semaphore_signal(
            bsem, inc=1,
            device_id=((me + j) % N_DEV,),
            device_id_type=pl.DeviceIdType.MESH,
        )
    pl.semaphore_wait(bsem, N_DEV - 1)

    sends = []
    for j in range(1, N_DEV):
        r = pltpu.make_async_remote_copy(
            src_ref=disp_ref.at[j],
            dst_ref=recv_ref.at[N_DEV - j],
            send_sem=send_sem.at[j],
            recv_sem=recv_sem.at[N_DEV - j],
            device_id=((me + j) % N_DEV,),
            device_id_type=pl.DeviceIdType.MESH,
        )
        r.start()
        sends.append(r)

    shared_ref[...] = jnp.dot(
        x_ref[...], sw_ref[...], preferred_element_type=jnp.float32
    )
    for k in range(N_LOCAL_E):
        yout_ref[0, k] = jnp.dot(
            disp_ref[0, k], w_ref[k], preferred_element_type=jnp.float32
        ).astype(jnp.bfloat16)

    sends2 = []
    for j in range(1, N_DEV):
        recv_only = pltpu.make_async_remote_copy(
            src_ref=disp_ref.at[j],
            dst_ref=recv_ref.at[j],
            send_sem=send_sem.at[j],
            recv_sem=recv_sem.at[j],
            device_id=((me + j) % N_DEV,),
            device_id_type=pl.DeviceIdType.MESH,
        )
        recv_only.wait_recv()
        for k in range(N_LOCAL_E):
            recv_ref[j, k] = jnp.dot(
                recv_ref[j, k], w_ref[k], preferred_element_type=jnp.float32
            ).astype(jnp.bfloat16)
        r2 = pltpu.make_async_remote_copy(
            src_ref=recv_ref.at[j],
            dst_ref=yout_ref.at[N_DEV - j],
            send_sem=send_sem2.at[j],
            recv_sem=recv_sem2.at[N_DEV - j],
            device_id=((me + j) % N_DEV,),
            device_id_type=pl.DeviceIdType.MESH,
        )
        r2.start()
        sends2.append(r2)

    for r in sends:
        r.wait_send()
    for r in sends2:
        r.wait_send()
    for j in range(1, N_DEV):
        recv2_only = pltpu.make_async_remote_copy(
            src_ref=recv_ref.at[j],
            dst_ref=yout_ref.at[j],
            send_sem=send_sem2.at[j],
            recv_sem=recv_sem2.at[j],
            device_id=((me + j) % N_DEV,),
            device_id_type=pl.DeviceIdType.MESH,
        )
        recv2_only.wait_recv()


def _cast_body(w_ref, o_ref):
    o_ref[...] = w_ref[...].astype(jnp.bfloat16)


def _combine_body(oh_ref, y_ref, sh_ref, p_ref, out_ref):
    t = jnp.dot(oh_ref[...], y_ref[...], preferred_element_type=jnp.float32)
    out_ref[...] = sh_ref[...] + p_ref[...] * t


def kernel(x, router_W, route_idx, expert_W, shared_W):
    n_tok, d = x.shape
    me = lax.axis_index("i")

    e = route_idx[:, 0]
    oh32 = (e[:, None] == jnp.arange(N_EXPERTS, dtype=e.dtype)[None, :])
    ohf = oh32.astype(jnp.float32)
    rank = jnp.sum(ohf * (jnp.cumsum(ohf, axis=0) - 1.0), axis=1).astype(jnp.int32)
    rot = ((e // N_LOCAL_E) - me) % N_DEV
    slot = rot * (N_LOCAL_E * CAP) + (e % N_LOCAL_E) * CAP + rank
    slot = jnp.where(rank < CAP, slot, jnp.int32(1 << 30))

    srange = jnp.arange(N_SLOTS, dtype=jnp.int32)
    onehotT = (srange[:, None] == slot[None, :]).astype(jnp.bfloat16)
    onehot = (slot[:, None] == srange[None, :]).astype(jnp.bfloat16)

    sw_bf = shared_W.astype(jnp.bfloat16)
    w_bf = pl.pallas_call(
        _cast_body,
        grid=(N_LOCAL_E, 8),
        in_specs=[
            pl.BlockSpec((1, d // 8, d), lambda i, j: (i, j, 0)),
        ],
        out_specs=pl.BlockSpec((1, d // 8, d), lambda i, j: (i, j, 0)),
        out_shape=jax.ShapeDtypeStruct(expert_W.shape, jnp.bfloat16),
    )(expert_W)

    disp_flat, p, x_bf = pl.pallas_call(
        _prep_body,
        out_shape=(
            jax.ShapeDtypeStruct((N_SLOTS, d), jnp.bfloat16),
            jax.ShapeDtypeStruct((n_tok, 1), jnp.float32),
            jax.ShapeDtypeStruct((n_tok, d), jnp.bfloat16),
        ),
        in_specs=[pl.BlockSpec(memory_space=pltpu.VMEM)] * 4,
        out_specs=(pl.BlockSpec(memory_space=pltpu.VMEM),) * 3,
        compiler_params=pltpu.CompilerParams(vmem_limit_bytes=_VMEM),
    )(x, router_W, route_idx, onehotT)
    disp = disp_flat.reshape(N_DEV, N_LOCAL_E, CAP, d)

    y_recv, shared_out = pl.pallas_call(
        _moe_body,
        out_shape=(
            jax.ShapeDtypeStruct((N_DEV, N_LOCAL_E, CAP, d), jnp.bfloat16),
            jax.ShapeDtypeStruct((n_tok, d), jnp.float32),
        ),
        in_specs=[pl.BlockSpec(memory_space=pltpu.VMEM)] * 4,
        out_specs=(pl.BlockSpec(memory_space=pltpu.VMEM),) * 2,
        scratch_shapes=[
            pltpu.VMEM((N_DEV, N_LOCAL_E, CAP, d), jnp.bfloat16),
            pltpu.SemaphoreType.DMA((N_DEV,)),
            pltpu.SemaphoreType.DMA((N_DEV,)),
            pltpu.SemaphoreType.DMA((N_DEV,)),
            pltpu.SemaphoreType.DMA((N_DEV,)),
        ],
        compiler_params=pltpu.CompilerParams(
            collective_id=0,
            vmem_limit_bytes=_VMEM,
        ),
    )(disp, x_bf, sw_bf, w_bf)

    out = pl.pallas_call(
        _combine_body,
        out_shape=jax.ShapeDtypeStruct((n_tok, d), jnp.float32),
        in_specs=[pl.BlockSpec(memory_space=pltpu.VMEM)] * 4,
        out_specs=pl.BlockSpec(memory_space=pltpu.VMEM),
        compiler_params=pltpu.CompilerParams(vmem_limit_bytes=_VMEM),
    )(onehot, y_recv.reshape(N_SLOTS, d), shared_out, p)
    return out


# device time: 241328 ns/iter; 1.0848x vs baseline; 1.0848x over previous
import jax
import jax.numpy as jnp
from jax import lax
from jax.experimental import pallas as pl
from jax.experimental.pallas import tpu as pltpu

N_DEV = 4
N_LOCAL_E = 8
N_EXPERTS = N_DEV * N_LOCAL_E
CAP = 128
N_SLOTS = N_DEV * N_LOCAL_E * CAP

_VMEM = 64 * 1024 * 1024


def _prep_body(x_ref, rw_ref, ri_ref, slot_ref, disp_ref, p_ref, xbf_ref):
    xb = x_ref[...].astype(jnp.bfloat16)
    xbf_ref[...] = xb
    srange = lax.broadcasted_iota(jnp.int32, (slot_ref.shape[0], N_SLOTS), 1)
    onehot = (srange == slot_ref[...]).astype(jnp.bfloat16)
    disp_ref[...] = lax.dot_general(
        onehot, xb, (((0,), (0,)), ((), ())),
        preferred_element_type=jnp.float32,
    ).astype(jnp.bfloat16)

    scores = jnp.dot(x_ref[...], rw_ref[...], preferred_element_type=jnp.float32)
    m = jnp.max(scores, axis=-1, keepdims=True)
    ex = jnp.exp(scores - m)
    probs = ex / jnp.sum(ex, axis=-1, keepdims=True)
    eid = lax.broadcasted_iota(jnp.int32, scores.shape, 1)
    sel = (eid == ri_ref[...]).astype(jnp.float32)
    p_ref[...] = jnp.sum(probs * sel, axis=-1, keepdims=True)


def _moe_body(disp_ref, x_ref, sw_ref, w_ref,
              yout_ref, shared_ref,
              recv_ref, send_sem, recv_sem, send_sem2, recv_sem2):
    me = lax.axis_index("i")

    bsem = pltpu.get_barrier_semaphore()
    for j in range(1, N_DEV):
        pl.semaphore_signal(
            bsem, inc=1,
            device_id=((me + j) % N_DEV,),
            device_id_type=pl.DeviceIdType.MESH,
        )
    pl.semaphore_wait(bsem, N_DEV - 1)

    sends = []
    for j in range(1, N_DEV):
        r = pltpu.make_async_remote_copy(
            src_ref=disp_ref.at[j],
            dst_ref=recv_ref.at[N_DEV - j],
            send_sem=send_sem.at[j],
            recv_sem=recv_sem.at[N_DEV - j],
            device_id=((me + j) % N_DEV,),
            device_id_type=pl.DeviceIdType.MESH,
        )
        r.start()
        sends.append(r)

    shared_ref[...] = jnp.dot(
        x_ref[...], sw_ref[...], preferred_element_type=jnp.float32
    )
    for k in range(N_LOCAL_E):
        yout_ref[0, k] = jnp.dot(
            disp_ref[0, k], w_ref[k], preferred_element_type=jnp.float32
        ).astype(jnp.bfloat16)

    sends2 = []
    for j in range(1, N_DEV):
        recv_only = pltpu.make_async_remote_copy(
            src_ref=disp_ref.at[j],
            dst_ref=recv_ref.at[j],
            send_sem=send_sem.at[j],
            recv_sem=recv_sem.at[j],
            device_id=((me + j) % N_DEV,),
            device_id_type=pl.DeviceIdType.MESH,
        )
        recv_only.wait_recv()
        for k in range(N_LOCAL_E):
            recv_ref[j, k] = jnp.dot(
                recv_ref[j, k], w_ref[k], preferred_element_type=jnp.float32
            ).astype(jnp.bfloat16)
        r2 = pltpu.make_async_remote_copy(
            src_ref=recv_ref.at[j],
            dst_ref=yout_ref.at[N_DEV - j],
            send_sem=send_sem2.at[j],
            recv_sem=recv_sem2.at[N_DEV - j],
            device_id=((me + j) % N_DEV,),
            device_id_type=pl.DeviceIdType.MESH,
        )
        r2.start()
        sends2.append(r2)

    for r in sends:
        r.wait_send()
    for r in sends2:
        r.wait_send()
    for j in range(1, N_DEV):
        recv2_only = pltpu.make_async_remote_copy(
            src_ref=recv_ref.at[j],
            dst_ref=yout_ref.at[j],
            send_sem=send_sem2.at[j],
            recv_sem=recv_sem2.at[j],
            device_id=((me + j) % N_DEV,),
            device_id_type=pl.DeviceIdType.MESH,
        )
        recv2_only.wait_recv()


def _cast_body(w_ref, o_ref):
    o_ref[...] = w_ref[...].astype(jnp.bfloat16)


def _combine_body(slot_ref, y_ref, sh_ref, p_ref, out_ref):
    srange = lax.broadcasted_iota(jnp.int32, (slot_ref.shape[0], N_SLOTS), 1)
    onehot = (srange == slot_ref[...]).astype(jnp.bfloat16)
    t = jnp.dot(onehot, y_ref[...], preferred_element_type=jnp.float32)
    out_ref[...] = sh_ref[...] + p_ref[...] * t


def kernel(x, router_W, route_idx, expert_W, shared_W):
    n_tok, d = x.shape
    me = lax.axis_index("i")

    e = route_idx[:, 0]
    oh32 = (e[:, None] == jnp.arange(N_EXPERTS, dtype=e.dtype)[None, :])
    ohf = oh32.astype(jnp.float32)
    rank = jnp.sum(ohf * (jnp.cumsum(ohf, axis=0) - 1.0), axis=1).astype(jnp.int32)
    rot = ((e // N_LOCAL_E) - me) % N_DEV
    slot = rot * (N_LOCAL_E * CAP) + (e % N_LOCAL_E) * CAP + rank
    slot = jnp.where(rank < CAP, slot, jnp.int32(1 << 30))
    slot = slot[:, None]

    sw_bf = shared_W.astype(jnp.bfloat16)
    w_bf = pl.pallas_call(
        _cast_body,
        grid=(N_LOCAL_E, 8),
        in_specs=[
            pl.BlockSpec((1, d // 8, d), lambda i, j: (i, j, 0)),
        ],
        out_specs=pl.BlockSpec((1, d // 8, d), lambda i, j: (i, j, 0)),
        out_shape=jax.ShapeDtypeStruct(expert_W.shape, jnp.bfloat16),
    )(expert_W)

    disp_flat, p, x_bf = pl.pallas_call(
        _prep_body,
        out_shape=(
            jax.ShapeDtypeStruct((N_SLOTS, d), jnp.bfloat16),
            jax.ShapeDtypeStruct((n_tok, 1), jnp.float32),
            jax.ShapeDtypeStruct((n_tok, d), jnp.bfloat16),
        ),
        in_specs=[pl.BlockSpec(memory_space=pltpu.VMEM)] * 4,
        out_specs=(pl.BlockSpec(memory_space=pltpu.VMEM),) * 3,
        compiler_params=pltpu.CompilerParams(vmem_limit_bytes=_VMEM),
    )(x, router_W, route_idx, slot)
    disp = disp_flat.reshape(N_DEV, N_LOCAL_E, CAP, d)

    y_recv, shared_out = pl.pallas_call(
        _moe_body,
        out_shape=(
            jax.ShapeDtypeStruct((N_DEV, N_LOCAL_E, CAP, d), jnp.bfloat16),
            jax.ShapeDtypeStruct((n_tok, d), jnp.float32),
        ),
        in_specs=[pl.BlockSpec(memory_space=pltpu.VMEM)] * 4,
        out_specs=(pl.BlockSpec(memory_space=pltpu.VMEM),) * 2,
        scratch_shapes=[
            pltpu.VMEM((N_DEV, N_LOCAL_E, CAP, d), jnp.bfloat16),
            pltpu.SemaphoreType.DMA((N_DEV,)),
            pltpu.SemaphoreType.DMA((N_DEV,)),
            pltpu.SemaphoreType.DMA((N_DEV,)),
            pltpu.SemaphoreType.DMA((N_DEV,)),
        ],
        compiler_params=pltpu.CompilerParams(
            collective_id=0,
            vmem_limit_bytes=_VMEM,
        ),
    )(disp, x_bf, sw_bf, w_bf)

    out = pl.pallas_call(
        _combine_body,
        out_shape=jax.ShapeDtypeStruct((n_tok, d), jnp.float32),
        in_specs=[pl.BlockSpec(memory_space=pltpu.VMEM)] * 4,
        out_specs=pl.BlockSpec(memory_space=pltpu.VMEM),
        compiler_params=pltpu.CompilerParams(vmem_limit_bytes=_VMEM),
    )(slot, y_recv.reshape(N_SLOTS, d), shared_out, p)
    return out
